# Initial kernel scaffold; baseline (speedup 1.0000x reference)
import jax
import jax.numpy as jnp
from jax.experimental import pallas as pl


def kernel(x, edge_index, lin1_w, lin1_b, conv_w, lin2_w, lin2_b):
    raise NotImplementedError("write your pallas kernel here")


# speedup vs baseline: 10.0402x; 10.0402x over previous
import functools
import math

import jax
import jax.numpy as jnp
from jax import lax
from jax.experimental import pallas as pl
from jax.experimental.pallas import tpu as pltpu
from jax.experimental.pallas import tpu_sc as plsc

ALPHA = 0.1
THETA = 0.5

NC = 2
NS = 16


def _edge_chunking(E):
    NW = NC * NS
    assert E % NW == 0, E
    per_w = E // NW
    K = max(k for k in range(8, 129, 8) if per_w % k == 0)
    return per_w, K, per_w // K


def _row_range(s, N):
    rows = -(-N // NS)
    rows = -(-rows // 8) * 8
    r0 = pl.multiple_of(jnp.minimum(s * rows, N - rows), 8)
    return r0, rows


def _make_hist(N, E):
    per_w, K, n_chunks = _edge_chunking(E)
    mesh = plsc.VectorSubcoreMesh(core_axis_name="c", subcore_axis_name="s")

    @functools.partial(
        pl.kernel,
        mesh=mesh,
        out_type=jax.ShapeDtypeStruct((NC, N, 16), jnp.float32),
        scratch_types=[
            pltpu.VMEM((K,), jnp.int32),
            pltpu.VMEM((K, 16), jnp.float32),
            pltpu.VMEM_SHARED((N, 16), jnp.float32),
        ],
    )
    def hist(col_hbm, zero_hbm, ones_hbm, out_hbm, coli_v, ones_v, acc_sh):
        c = lax.axis_index("c")
        s = lax.axis_index("s")
        wid = s * NC + c
        r0, nr = _row_range(s, N)
        pltpu.sync_copy(ones_hbm, ones_v)
        pltpu.sync_copy(zero_hbm.at[pl.ds(r0, nr)], acc_sh.at[pl.ds(r0, nr)])
        plsc.subcore_barrier()

        def step(i, carry):
            off = pl.multiple_of(wid * per_w + i * K, 8)
            pltpu.sync_copy(col_hbm.at[pl.ds(off, K)], coli_v)
            pltpu.sync_copy(ones_v, acc_sh.at[coli_v], add=True)
            return carry

        lax.fori_loop(0, n_chunks, step, 0)
        plsc.subcore_barrier()
        pltpu.sync_copy(acc_sh.at[pl.ds(r0, nr)],
                        out_hbm.at[c, pl.ds(r0, nr)])

    return hist


def _make_prop(N, D, E):
    per_w, K, n_chunks = _edge_chunking(E)
    mesh = plsc.VectorSubcoreMesh(core_axis_name="c", subcore_axis_name="s")

    @functools.partial(
        pl.kernel,
        mesh=mesh,
        out_type=jax.ShapeDtypeStruct((NC, N, D), jnp.float32),
        scratch_types=[
            pltpu.VMEM((K,), jnp.int32),
            pltpu.VMEM((K,), jnp.int32),
            pltpu.VMEM((K, D), jnp.float32),
            pltpu.VMEM_SHARED((N, D), jnp.float32),
            pltpu.SemaphoreType.DMA,
        ],
    )
    def prop(g_hbm, row_hbm, col_hbm, zero_hbm, out_hbm,
             rowi_v, coli_v, rows_v, acc_sh, sem):
        c = lax.axis_index("c")
        s = lax.axis_index("s")
        wid = s * NC + c
        r0, nr = _row_range(s, N)
        pltpu.sync_copy(zero_hbm.at[pl.ds(r0, nr)], acc_sh.at[pl.ds(r0, nr)])
        plsc.subcore_barrier()

        def step(i, carry):
            off = pl.multiple_of(wid * per_w + i * K, 8)
            pltpu.sync_copy(row_hbm.at[pl.ds(off, K)], rowi_v)
            pltpu.sync_copy(col_hbm.at[pl.ds(off, K)], coli_v)
            pltpu.async_copy(g_hbm.at[rowi_v], rows_v, sem).wait()
            pltpu.sync_copy(rows_v, acc_sh.at[coli_v], add=True)
            return carry

        lax.fori_loop(0, n_chunks, step, 0)
        plsc.subcore_barrier()
        pltpu.sync_copy(acc_sh.at[pl.ds(r0, nr)],
                        out_hbm.at[c, pl.ds(r0, nr)])

    return prop


def _dinv_from_deg(degp_ref):
    deg = (jnp.sum(degp_ref[0], axis=-1, keepdims=True)
           + jnp.sum(degp_ref[1], axis=-1, keepdims=True) + 1.0)
    return lax.rsqrt(deg)


def _tc1_body(x_ref, w1_ref, b1_ref, degp_ref, h0_ref, g0_ref):
    h = jnp.dot(x_ref[...], w1_ref[...], preferred_element_type=jnp.float32)
    h = jnp.maximum(h + b1_ref[...], 0.0)
    h0_ref[...] = h
    g0_ref[...] = _dinv_from_deg(degp_ref) * h


def _layer_body(alpha, beta, degp_ref, s2_ref, h_ref, x0_ref, w_ref,
                h1_ref, g1_ref):
    dinv = _dinv_from_deg(degp_ref)
    s = s2_ref[0] + s2_ref[1]
    m = (1.0 - alpha) * (dinv * s + (dinv * dinv) * h_ref[...]) \
        + alpha * x0_ref[...]
    t = jnp.dot(m, w_ref[...], preferred_element_type=jnp.float32)
    hn = jnp.maximum((1.0 - beta) * m + beta * t, 0.0)
    h1_ref[...] = hn
    g1_ref[...] = dinv * hn


def _fin_body(h_ref, w2_ref, b2_ref, o_ref):
    o_ref[...] = (jnp.dot(h_ref[...], w2_ref[...],
                          preferred_element_type=jnp.float32) + b2_ref[...])


def kernel(x, edge_index, lin1_w, lin1_b, conv_w, lin2_w, lin2_b):
    N, D = x.shape
    L = conv_w.shape[0]
    ei = edge_index.astype(jnp.int32)
    row, col = ei[0], ei[1]
    E = row.shape[0]
    per_w, K, _ = _edge_chunking(E)

    z16 = jnp.zeros((N, 16), jnp.float32)
    zD = jnp.zeros((N, D), jnp.float32)
    ones = jnp.ones((K, 16), jnp.float32)

    degp = _make_hist(N, E)(col, z16, ones)
    prop = _make_prop(N, D, E)

    R = 2000 if N % 2000 == 0 else N
    nb = N // R
    full = pl.BlockSpec((R, D), lambda i: (i, 0))
    wspec = pl.BlockSpec((D, D), lambda i: (0, 0))
    bspec = pl.BlockSpec((1, D), lambda i: (0, 0))
    degspec = pl.BlockSpec((NC, R, 16), lambda i: (0, i, 0))
    s2spec = pl.BlockSpec((NC, R, D), lambda i: (0, i, 0))
    fDD = jax.ShapeDtypeStruct((N, D), jnp.float32)

    h0, g = pl.pallas_call(
        _tc1_body,
        grid=(nb,),
        in_specs=[full, wspec, bspec, degspec],
        out_specs=[full, full],
        out_shape=[fDD, fDD],
    )(x, lin1_w, lin1_b.reshape(1, D), degp)

    h = h0
    for layer in range(L):
        beta = math.log(THETA / (layer + 1) + 1.0)
        s2 = prop(g, row, col, zD)
        h, g = pl.pallas_call(
            functools.partial(_layer_body, ALPHA, beta),
            grid=(nb,),
            in_specs=[degspec, s2spec, full, full, wspec],
            out_specs=[full, full],
            out_shape=[fDD, fDD],
        )(degp, s2, h, h0, conv_w[layer])

    out = pl.pallas_call(
        _fin_body,
        grid=(nb,),
        in_specs=[full, wspec, bspec],
        out_specs=full,
        out_shape=fDD,
    )(h, lin2_w, lin2_b.reshape(1, D))
    return out
